# baseline (speedup 1.0000x reference)
import functools

import numpy as np
import jax
import jax.numpy as jnp
from jax import lax
from jax.experimental import pallas as pl
from jax.experimental.pallas import tpu as pltpu
from jax.experimental.pallas import tpu_sc as plsc


def _pos_table(t, dim):
    omega = np.arange(dim // 2, dtype=np.float64)
    omega = 1.0 / (10000.0 ** (omega / (dim / 2.0)))
    out = np.einsum("p,d->pd", np.arange(t, dtype=np.float64), omega)
    return np.concatenate([np.sin(out), np.cos(out)], axis=-1).astype(np.float32)


def _month_table(d_hid):
    angles = np.arange(0, 13) / (12.0 / (2.0 * np.pi))
    sin_t = np.sin(np.stack([angles] * (d_hid // 2), axis=-1))
    cos_t = np.cos(np.stack([angles] * (d_hid // 2), axis=-1))
    return np.concatenate([sin_t[:-1], cos_t[:-1]], axis=-1).astype(np.float32)


@functools.lru_cache(maxsize=None)
def _make_sc_kernel(n_rows, d, t_len, n_bands):
    info = plsc.get_sparse_core_info()
    nc, ns = info.num_cores, info.num_subcores
    nw = nc * ns
    n = d // 4
    n_groups = n_rows // n_bands
    gpw = n_groups // nw
    rpw = n_rows // nw
    assert n_groups % nw == 0 and gpw % 2 == 0
    assert gpw % t_len == 0 or (gpw * nw) % t_len == 0
    nv = n // 16
    mesh = plsc.VectorSubcoreMesh(core_axis_name="c", subcore_axis_name="s")

    @functools.partial(
        pl.kernel,
        mesh=mesh,
        out_type=jax.ShapeDtypeStruct((n_rows, d), jnp.float32),
        scratch_types=[
            pltpu.VMEM((gpw,), jnp.int32),
            pltpu.VMEM((gpw, n), jnp.float32),
            pltpu.VMEM((n_bands, n), jnp.float32),
            pltpu.VMEM((t_len, n), jnp.float32),
            pltpu.VMEM((2, n_bands, d), jnp.float32),
            pltpu.VMEM((2, n_bands, d), jnp.float32),
            pltpu.SemaphoreType.DMA,
            pltpu.SemaphoreType.DMA,
            pltpu.SemaphoreType.DMA,
            pltpu.SemaphoreType.DMA,
            pltpu.SemaphoreType.DMA,
        ],
    )
    def body(tok_hbm, midx_hbm, ch_hbm, pos_hbm, mtab_hbm, out_hbm,
             idx_v, mon_v, ch_v, pos_v, inb, outb,
             sem_m, sem_i0, sem_i1, sem_o0, sem_o1):
        sem_i = (sem_i0, sem_i1)
        sem_o = (sem_o0, sem_o1)
        wid = lax.axis_index("s") * nc + lax.axis_index("c")
        g0 = wid * gpw
        r0 = wid * rpw
        pltpu.sync_copy(midx_hbm.at[pl.ds(g0, gpw)], idx_v)
        pltpu.make_async_copy(mtab_hbm.at[idx_v], mon_v, sem_m).start()
        pltpu.sync_copy(ch_hbm, ch_v)
        pltpu.sync_copy(pos_hbm, pos_v)
        pltpu.make_async_copy(mtab_hbm.at[idx_v], mon_v, sem_m).wait()
        for j in range(2):
            pltpu.make_async_copy(
                tok_hbm.at[pl.ds(r0 + j * n_bands, n_bands)], inb.at[j], sem_i[j]
            ).start()

        def chunk(c, j):
            row = r0 + c * n_bands
            t_idx = lax.rem(g0 + c, t_len)
            pltpu.make_async_copy(
                tok_hbm.at[pl.ds(row, n_bands)], inb.at[j], sem_i[j]
            ).wait()

            @pl.when(c >= 2)
            def _wait_prev_scatter():
                pltpu.make_async_copy(
                    outb.at[j], out_hbm.at[pl.ds(row, n_bands)], sem_o[j]
                ).wait()

            pm_vec = ([pos_v[t_idx, pl.ds(v * 16, 16)] for v in range(nv)]
                      + [mon_v[c, pl.ds(v * 16, 16)] for v in range(nv)])
            for r in range(n_bands):
                for v in range(nv):
                    off = v * 16
                    outb[j, r, pl.ds(off, 16)] = (
                        inb[j, r, pl.ds(off, 16)] + ch_v[r, pl.ds(off, 16)])
                    outb[j, r, pl.ds(n + off, 16)] = (
                        inb[j, r, pl.ds(n + off, 16)] + pm_vec[v])
                    outb[j, r, pl.ds(2 * n + off, 16)] = (
                        inb[j, r, pl.ds(2 * n + off, 16)] + pm_vec[nv + v])
                    outb[j, r, pl.ds(3 * n + off, 16)] = inb[j, r, pl.ds(3 * n + off, 16)]

            pltpu.make_async_copy(
                outb.at[j], out_hbm.at[pl.ds(row, n_bands)], sem_o[j]
            ).start()

            @pl.when(c + 2 < gpw)
            def _prefetch_next():
                pltpu.make_async_copy(
                    tok_hbm.at[pl.ds(row + 2 * n_bands, n_bands)], inb.at[j], sem_i[j]
                ).start()

        def loop_body(g, carry):
            for j in range(2):
                chunk(g * 2 + j, j)
            return carry

        lax.fori_loop(0, gpw // 2, loop_body, 0)
        for j in range(2):
            pltpu.make_async_copy(
                outb.at[j], out_hbm.at[pl.ds(r0, n_bands)], sem_o[j]
            ).wait()

    return body


def kernel(tokens, timestamps, ch_embed, patch_size):
    b, t, s, d = tokens.shape
    n = d // 4
    pos = jnp.asarray(_pos_table(t, n))
    mtab = jnp.asarray(_month_table(n))
    months = timestamps[:, :, 1].reshape(-1)
    tok_rows = tokens.reshape(b * t * s, d)
    out = _make_sc_kernel(b * t * s, d, t, s)(tok_rows, months, ch_embed, pos, mtab)
    return out.reshape(b, t, s, d)


# speedup vs baseline: 1.7156x; 1.7156x over previous
import functools

import numpy as np
import jax
import jax.numpy as jnp
from jax import lax
from jax.experimental import pallas as pl
from jax.experimental.pallas import tpu as pltpu
from jax.experimental.pallas import tpu_sc as plsc


def _pos_table(t, dim):
    omega = np.arange(dim // 2, dtype=np.float64)
    omega = 1.0 / (10000.0 ** (omega / (dim / 2.0)))
    out = np.einsum("p,d->pd", np.arange(t, dtype=np.float64), omega)
    return np.concatenate([np.sin(out), np.cos(out)], axis=-1).astype(np.float32)


def _month_table(d_hid):
    angles = np.arange(0, 13) / (12.0 / (2.0 * np.pi))
    sin_t = np.sin(np.stack([angles] * (d_hid // 2), axis=-1))
    cos_t = np.cos(np.stack([angles] * (d_hid // 2), axis=-1))
    return np.concatenate([sin_t[:-1], cos_t[:-1]], axis=-1).astype(np.float32)


@functools.lru_cache(maxsize=None)
def _make_sc_gather(n_rows, d):
    info = plsc.get_sparse_core_info()
    nc, ns = info.num_cores, info.num_subcores
    nw = nc * ns
    per_w = n_rows // nw
    assert n_rows % nw == 0 and per_w % 8 == 0
    mesh = plsc.VectorSubcoreMesh(core_axis_name="c", subcore_axis_name="s")

    @functools.partial(
        pl.kernel,
        mesh=mesh,
        out_type=jax.ShapeDtypeStruct((n_rows, d), jnp.float32),
        scratch_types=[
            pltpu.VMEM((per_w,), jnp.int32),
            pltpu.VMEM((per_w, d), jnp.float32),
            pltpu.SemaphoreType.DMA,
            pltpu.SemaphoreType.DMA,
            pltpu.SemaphoreType.DMA,
            pltpu.SemaphoreType.DMA,
        ],
    )
    def gather(table_hbm, idx_hbm, out_hbm, idx_v, rows_v, sem_a, sem_b, sem_c, sem_d):
        wid = lax.axis_index("s") * nc + lax.axis_index("c")
        base = wid * per_w
        half = per_w // 2
        pltpu.sync_copy(idx_hbm.at[pl.ds(base, per_w)], idx_v)
        g0 = pltpu.make_async_copy(
            table_hbm.at[idx_v.at[pl.ds(0, half)]], rows_v.at[pl.ds(0, half)], sem_a)
        g1 = pltpu.make_async_copy(
            table_hbm.at[idx_v.at[pl.ds(half, half)]], rows_v.at[pl.ds(half, half)], sem_b)
        g0.start()
        g1.start()
        g0.wait()
        s0 = pltpu.make_async_copy(
            rows_v.at[pl.ds(0, half)], out_hbm.at[pl.ds(base, half)], sem_c)
        s0.start()
        g1.wait()
        s1 = pltpu.make_async_copy(
            rows_v.at[pl.ds(half, half)], out_hbm.at[pl.ds(base + half, half)], sem_d)
        s1.start()
        s0.wait()
        s1.wait()

    return gather


def _tc_body(tok_ref, mon_ref, pos_ref, ch_ref, out_ref):
    n = ch_ref.shape[-1]
    tok = tok_ref[...]
    out_ref[..., 0:n] = tok[..., 0:n] + ch_ref[...][None, None, :, :]
    out_ref[..., n:2 * n] = tok[..., n:2 * n] + pos_ref[...][None, :, None, :]
    out_ref[..., 2 * n:3 * n] = tok[..., 2 * n:3 * n] + mon_ref[...][:, :, None, :]
    out_ref[..., 3 * n:] = tok[..., 3 * n:]


def kernel(tokens, timestamps, ch_embed, patch_size):
    b, t, s, d = tokens.shape
    n = d // 4
    pos = jnp.asarray(_pos_table(t, n))
    mtab = jnp.asarray(_month_table(n))
    months = timestamps[:, :, 1].reshape(-1)
    month_e = _make_sc_gather(b * t, n)(mtab, months).reshape(b, t, n)
    bb = 8
    return pl.pallas_call(
        _tc_body,
        grid=(b // bb,),
        in_specs=[
            pl.BlockSpec((bb, t, s, d), lambda i: (i, 0, 0, 0)),
            pl.BlockSpec((bb, t, n), lambda i: (i, 0, 0)),
            pl.BlockSpec((t, n), lambda i: (0, 0)),
            pl.BlockSpec((s, n), lambda i: (0, 0)),
        ],
        out_specs=pl.BlockSpec((bb, t, s, d), lambda i: (i, 0, 0, 0)),
        out_shape=jax.ShapeDtypeStruct((b, t, s, d), jnp.float32),
    )(tokens, month_e, pos, ch_embed)


# speedup vs baseline: 1.7255x; 1.0058x over previous
import functools

import numpy as np
import jax
import jax.numpy as jnp
from jax import lax
from jax.experimental import pallas as pl
from jax.experimental.pallas import tpu as pltpu
from jax.experimental.pallas import tpu_sc as plsc


def _pos_table(t, dim):
    omega = np.arange(dim // 2, dtype=np.float64)
    omega = 1.0 / (10000.0 ** (omega / (dim / 2.0)))
    out = np.einsum("p,d->pd", np.arange(t, dtype=np.float64), omega)
    return np.concatenate([np.sin(out), np.cos(out)], axis=-1).astype(np.float32)


def _month_table(d_hid):
    angles = np.arange(0, 13) / (12.0 / (2.0 * np.pi))
    sin_t = np.sin(np.stack([angles] * (d_hid // 2), axis=-1))
    cos_t = np.cos(np.stack([angles] * (d_hid // 2), axis=-1))
    return np.concatenate([sin_t[:-1], cos_t[:-1]], axis=-1).astype(np.float32)


@functools.lru_cache(maxsize=None)
def _make_sc_gather(n_rows, d):
    info = plsc.get_sparse_core_info()
    nc, ns = info.num_cores, info.num_subcores
    nw = nc * ns
    per_w = n_rows // nw
    assert n_rows % nw == 0 and per_w % 8 == 0
    mesh = plsc.VectorSubcoreMesh(core_axis_name="c", subcore_axis_name="s")

    @functools.partial(
        pl.kernel,
        mesh=mesh,
        out_type=jax.ShapeDtypeStruct((n_rows, d), jnp.float32),
        scratch_types=[
            pltpu.VMEM((per_w,), jnp.int32),
            pltpu.VMEM((per_w, d), jnp.float32),
            pltpu.SemaphoreType.DMA,
        ],
    )
    def gather(table_hbm, idx_hbm, out_hbm, idx_v, rows_v, sem):
        wid = lax.axis_index("s") * nc + lax.axis_index("c")
        base = wid * per_w
        pltpu.sync_copy(idx_hbm.at[pl.ds(base, per_w)], idx_v)
        pltpu.async_copy(table_hbm.at[idx_v], rows_v, sem).wait()
        pltpu.sync_copy(rows_v, out_hbm.at[pl.ds(base, per_w)])

    return gather


def _tc_body(tok_ref, mon_ref, pos_ref, ch_ref, out_ref):
    n = ch_ref.shape[-1]
    tok = tok_ref[...]
    out_ref[..., 0:n] = tok[..., 0:n] + ch_ref[...][None, None, :, :]
    out_ref[..., n:2 * n] = tok[..., n:2 * n] + pos_ref[...][None, :, None, :]
    out_ref[..., 2 * n:3 * n] = tok[..., 2 * n:3 * n] + mon_ref[...][:, :, None, :]
    out_ref[..., 3 * n:] = tok[..., 3 * n:]


def kernel(tokens, timestamps, ch_embed, patch_size):
    b, t, s, d = tokens.shape
    n = d // 4
    pos = jnp.asarray(_pos_table(t, n))
    mtab = jnp.asarray(_month_table(n))
    months = timestamps[:, :, 1].reshape(-1)
    month_e = _make_sc_gather(b * t, n)(mtab, months).reshape(b, t, n)
    bb = 8
    return pl.pallas_call(
        _tc_body,
        grid=(b // bb,),
        in_specs=[
            pl.BlockSpec((bb, t, s, d), lambda i: (i, 0, 0, 0)),
            pl.BlockSpec((bb, t, n), lambda i: (i, 0, 0)),
            pl.BlockSpec((t, n), lambda i: (0, 0)),
            pl.BlockSpec((s, n), lambda i: (0, 0)),
        ],
        out_specs=pl.BlockSpec((bb, t, s, d), lambda i: (i, 0, 0, 0)),
        out_shape=jax.ShapeDtypeStruct((b, t, s, d), jnp.float32),
    )(tokens, month_e, pos, ch_embed)
